# Initial kernel scaffold; baseline (speedup 1.0000x reference)
import jax
import jax.numpy as jnp
from jax.experimental import pallas as pl


def kernel(input_features, input_ids, expert_centroids, ln_scale, ln_bias, W1, b1, W2, b2):
    raise NotImplementedError("write your pallas kernel here")


# speedup vs baseline: 2.1380x; 2.1380x over previous
import functools

import jax
import jax.numpy as jnp
from jax import lax
from jax.experimental import pallas as pl
from jax.experimental.pallas import tpu as pltpu
from jax.experimental.pallas import tpu_sc as plsc

_E, _D, _F = 8, 1024, 4096
_T = 2048
_BT = 128
_G = _T // _BT + _E
_TP = _G * _BT
_NW = 32
_TPW = _T // _NW
_GPAD = 128


def _route_body(x_ref, c_ref, dest_ref, alpha_ref, bexp_ref):
    x = x_ref[...]
    c = c_ref[...]
    aff = lax.dot_general(
        x, c, (((1,), (1,)), ((), ())),
        preferred_element_type=jnp.float32,
        precision=lax.Precision.DEFAULT)
    mx = jnp.max(aff, axis=1, keepdims=True)
    alpha = 1.0 / (1.0 + jnp.exp(-mx))
    alpha_ref[...] = jnp.broadcast_to(alpha, (_T, 128))
    eid = lax.broadcasted_iota(jnp.int32, (_T, _E), 1)
    idx = jnp.min(jnp.where(aff == mx, eid, _E), axis=1, keepdims=True)
    onehot = (eid == idx).astype(jnp.float32)
    tri = (lax.broadcasted_iota(jnp.int32, (_T, _T), 1)
           <= lax.broadcasted_iota(jnp.int32, (_T, _T), 0)).astype(jnp.float32)
    cum = lax.dot_general(tri, onehot, (((1,), (0,)), ((), ())),
                          preferred_element_type=jnp.float32)
    counts = cum[_T - 1:_T, :]
    rank = jnp.sum(cum * onehot, axis=1, keepdims=True) - 1.0
    nblk = jnp.ceil(counts * (1.0 / _BT))
    nb = jnp.broadcast_to(nblk, (_E, _E))
    strict_lo = (lax.broadcasted_iota(jnp.int32, (_E, _E), 1)
                 < lax.broadcasted_iota(jnp.int32, (_E, _E), 0)).astype(jnp.float32)
    blk_start = jnp.sum(nb * strict_lo, axis=1, keepdims=True)
    pad_start = blk_start * float(_BT)
    dest_base = lax.dot_general(onehot, pad_start, (((1,), (0,)), ((), ())),
                                preferred_element_type=jnp.float32)
    dest_ref[...] = (dest_base + rank).astype(jnp.int32)
    ge = (jnp.broadcast_to(blk_start, (_E, _GPAD))
          <= lax.broadcasted_iota(jnp.int32, (_E, _GPAD), 1).astype(jnp.float32)
          ).astype(jnp.float32)
    bexp_ref[...] = jnp.sum(ge, axis=0, keepdims=True).astype(jnp.int32) - 1


def _ffn_body(bexp_ref, xs_ref, al_ref, lns_ref, lnb_ref,
              w1_ref, b1_ref, w2_ref, b2_ref, y_ref):
    del bexp_ref
    xs = xs_ref[...]
    mu = jnp.mean(xs, axis=1, keepdims=True)
    var = jnp.mean((xs - mu) * (xs - mu), axis=1, keepdims=True)
    hn = (xs - mu) * lax.rsqrt(var + 1e-5) * lns_ref[0] + lnb_ref[0]
    h = jnp.maximum(
        lax.dot_general(hn.astype(jnp.bfloat16), w1_ref[0],
                        (((1,), (0,)), ((), ())),
                        preferred_element_type=jnp.float32) + b1_ref[0], 0.0)
    f = lax.dot_general(h.astype(jnp.bfloat16), w2_ref[0],
                        (((1,), (0,)), ((), ())),
                        preferred_element_type=jnp.float32) + b2_ref[0]
    y_ref[...] = xs + al_ref[:, :1] * f


def _dispatch_body(x_hbm, dest_hbm, alpha_hbm, xs_hbm, as_hbm,
                   idx_v, rows_v, al_v, sem1, sem2):
    wid = lax.axis_index("s") * 2 + lax.axis_index("c")
    base = wid * _TPW
    pltpu.sync_copy(dest_hbm.at[pl.ds(base, _TPW)], idx_v)
    pltpu.sync_copy(x_hbm.at[pl.ds(base, _TPW)], rows_v)
    pltpu.sync_copy(alpha_hbm.at[pl.ds(base, _TPW)], al_v)
    cp1 = pltpu.async_copy(rows_v, xs_hbm.at[idx_v], sem1)
    cp2 = pltpu.async_copy(al_v, as_hbm.at[idx_v], sem2)
    cp1.wait()
    cp2.wait()


def _combine_body(dest_hbm, y_hbm, out_hbm, idx_v, rows_v, sem):
    wid = lax.axis_index("s") * 2 + lax.axis_index("c")
    base = wid * _TPW
    pltpu.sync_copy(dest_hbm.at[pl.ds(base, _TPW)], idx_v)
    pltpu.async_copy(y_hbm.at[idx_v], rows_v, sem).wait()
    pltpu.sync_copy(rows_v, out_hbm.at[pl.ds(base, _TPW)])


@functools.lru_cache(maxsize=None)
def _sc_calls():
    mesh = plsc.VectorSubcoreMesh(core_axis_name="c", subcore_axis_name="s")
    dispatch = pl.kernel(
        _dispatch_body,
        out_type=[jax.ShapeDtypeStruct((_TP, _D), jnp.float32),
                  jax.ShapeDtypeStruct((_TP, 128), jnp.float32)],
        mesh=mesh,
        scratch_types=[pltpu.VMEM((_TPW,), jnp.int32),
                       pltpu.VMEM((_TPW, _D), jnp.float32),
                       pltpu.VMEM((_TPW, 128), jnp.float32),
                       pltpu.SemaphoreType.DMA,
                       pltpu.SemaphoreType.DMA])
    combine = pl.kernel(
        _combine_body,
        out_type=jax.ShapeDtypeStruct((_T, _D), jnp.float32),
        mesh=mesh,
        scratch_types=[pltpu.VMEM((_TPW,), jnp.int32),
                       pltpu.VMEM((_TPW, _D), jnp.float32),
                       pltpu.SemaphoreType.DMA])
    return dispatch, combine


_route_call = pl.pallas_call(
    _route_body,
    out_shape=[jax.ShapeDtypeStruct((_T, 1), jnp.int32),
               jax.ShapeDtypeStruct((_T, 128), jnp.float32),
               jax.ShapeDtypeStruct((1, _GPAD), jnp.int32)],
)

_ffn_call = pl.pallas_call(
    _ffn_body,
    grid_spec=pltpu.PrefetchScalarGridSpec(
        num_scalar_prefetch=1,
        grid=(_G,),
        in_specs=[
            pl.BlockSpec((_BT, _D), lambda g, be: (g, 0)),
            pl.BlockSpec((_BT, 128), lambda g, be: (g, 0)),
            pl.BlockSpec((1, 1, _D), lambda g, be: (be[g], 0, 0)),
            pl.BlockSpec((1, 1, _D), lambda g, be: (be[g], 0, 0)),
            pl.BlockSpec((1, _D, _F), lambda g, be: (be[g], 0, 0)),
            pl.BlockSpec((1, 1, _F), lambda g, be: (be[g], 0, 0)),
            pl.BlockSpec((1, _F, _D), lambda g, be: (be[g], 0, 0)),
            pl.BlockSpec((1, 1, _D), lambda g, be: (be[g], 0, 0)),
        ],
        out_specs=pl.BlockSpec((_BT, _D), lambda g, be: (g, 0)),
    ),
    out_shape=jax.ShapeDtypeStruct((_TP, _D), jnp.float32),
)


def kernel(input_features, input_ids, expert_centroids, ln_scale, ln_bias,
           W1, b1, W2, b2):
    s, b, d = input_features.shape
    x = input_features.reshape(s * b, d)
    dest2, alpha16, bexp2 = _route_call(x, expert_centroids)
    dest = dest2.reshape(_T)
    bexp = bexp2.reshape(_GPAD)
    dispatch, combine = _sc_calls()
    xs, als = dispatch(x, dest, alpha16)
    y = _ffn_call(bexp, xs, als,
                  ln_scale.reshape(_E, 1, _D), ln_bias.reshape(_E, 1, _D),
                  W1.astype(jnp.bfloat16), b1.reshape(_E, 1, _F),
                  W2.astype(jnp.bfloat16), b2.reshape(_E, 1, _D))
    out = combine(dest, y)
    return out.reshape(s, b, d)


# speedup vs baseline: 2.5795x; 1.2065x over previous
import functools

import jax
import jax.numpy as jnp
from jax import lax
from jax.experimental import pallas as pl
from jax.experimental.pallas import tpu as pltpu
from jax.experimental.pallas import tpu_sc as plsc

_E, _D, _F = 8, 1024, 4096
_T = 2048
_BT = 128
_G = _T // _BT + _E
_TP = _G * _BT
_NW = 32
_TPW = _T // _NW
_GPAD = 128
_NF = 2
_FC = _F // _NF


def _route_body(x_ref, c_ref, dest_ref, alpha_ref, bexp_ref):
    x = x_ref[...]
    c = c_ref[...]
    aff = lax.dot_general(
        x, c, (((1,), (1,)), ((), ())),
        preferred_element_type=jnp.float32,
        precision=lax.Precision.DEFAULT)
    mx = jnp.max(aff, axis=1, keepdims=True)
    alpha = 1.0 / (1.0 + jnp.exp(-mx))
    alpha_ref[...] = jnp.broadcast_to(alpha, (_T, 128))
    eid = lax.broadcasted_iota(jnp.int32, (_T, _E), 1)
    idx = jnp.min(jnp.where(aff == mx, eid, _E), axis=1, keepdims=True)
    onehot = (eid == idx).astype(jnp.float32)
    tri = (lax.broadcasted_iota(jnp.int32, (_T, _T), 1)
           <= lax.broadcasted_iota(jnp.int32, (_T, _T), 0)).astype(jnp.float32)
    cum = lax.dot_general(tri, onehot, (((1,), (0,)), ((), ())),
                          preferred_element_type=jnp.float32)
    counts = cum[_T - 1:_T, :]
    rank = jnp.sum(cum * onehot, axis=1, keepdims=True) - 1.0
    nblk = jnp.ceil(counts * (1.0 / _BT))
    nb = jnp.broadcast_to(nblk, (_E, _E))
    strict_lo = (lax.broadcasted_iota(jnp.int32, (_E, _E), 1)
                 < lax.broadcasted_iota(jnp.int32, (_E, _E), 0)).astype(jnp.float32)
    blk_start = jnp.sum(nb * strict_lo, axis=1, keepdims=True)
    pad_start = blk_start * float(_BT)
    dest_base = lax.dot_general(onehot, pad_start, (((1,), (0,)), ((), ())),
                                preferred_element_type=jnp.float32)
    dest_ref[...] = (dest_base + rank).astype(jnp.int32)
    ge = (jnp.broadcast_to(blk_start, (_E, _GPAD))
          <= lax.broadcasted_iota(jnp.int32, (_E, _GPAD), 1).astype(jnp.float32)
          ).astype(jnp.float32)
    bexp_ref[...] = jnp.sum(ge, axis=0, keepdims=True).astype(jnp.int32) - 1


def _ffn_body(bexp_ref, xs_ref, al_ref, lns_ref, lnb_ref,
              w1_ref, b1_ref, w2_ref, b2_ref, y_ref,
              acc_ref, w1b_ref, w2b_ref):
    f = pl.program_id(0)
    g = pl.program_id(1)
    e = bexp_ref[g]
    eprev = bexp_ref[jnp.maximum(g - 1, 0)]

    @pl.when((g == 0) | (e != eprev))
    def _cast_weights():
        w1b_ref[...] = w1_ref[0].astype(jnp.bfloat16)
        w2b_ref[...] = w2_ref[0].astype(jnp.bfloat16)

    xs = xs_ref[...]
    mu = jnp.mean(xs, axis=1, keepdims=True)
    var = jnp.mean((xs - mu) * (xs - mu), axis=1, keepdims=True)
    hn = (xs - mu) * lax.rsqrt(var + 1e-5) * lns_ref[0] + lnb_ref[0]
    h = jnp.maximum(
        lax.dot_general(hn.astype(jnp.bfloat16), w1b_ref[...],
                        (((1,), (0,)), ((), ())),
                        preferred_element_type=jnp.float32) + b1_ref[0], 0.0)
    part = lax.dot_general(h.astype(jnp.bfloat16), w2b_ref[...],
                           (((1,), (0,)), ((), ())),
                           preferred_element_type=jnp.float32)

    @pl.when(f == 0)
    def _store_partial():
        acc_ref[pl.ds(g * _BT, _BT), :] = part

    @pl.when((f > 0) & (f < _NF - 1))
    def _add_partial():
        acc_ref[pl.ds(g * _BT, _BT), :] += part

    @pl.when(f == _NF - 1)
    def _finish():
        y_ref[...] = xs + al_ref[:, :1] * (
            acc_ref[pl.ds(g * _BT, _BT), :] + part + b2_ref[0])


def _dispatch_body(x_hbm, dest_hbm, alpha_hbm, xs_hbm, as_hbm,
                   idx_v, rows_v, al_v, sem1, sem2):
    wid = lax.axis_index("s") * 2 + lax.axis_index("c")
    base = wid * _TPW
    pltpu.sync_copy(dest_hbm.at[pl.ds(base, _TPW)], idx_v)
    pltpu.sync_copy(x_hbm.at[pl.ds(base, _TPW)], rows_v)
    pltpu.sync_copy(alpha_hbm.at[pl.ds(base, _TPW)], al_v)
    cp1 = pltpu.async_copy(rows_v, xs_hbm.at[idx_v], sem1)
    cp2 = pltpu.async_copy(al_v, as_hbm.at[idx_v], sem2)
    cp1.wait()
    cp2.wait()


def _combine_body(dest_hbm, y_hbm, out_hbm, idx_v, rows_v, sem):
    wid = lax.axis_index("s") * 2 + lax.axis_index("c")
    base = wid * _TPW
    pltpu.sync_copy(dest_hbm.at[pl.ds(base, _TPW)], idx_v)
    pltpu.async_copy(y_hbm.at[idx_v], rows_v, sem).wait()
    pltpu.sync_copy(rows_v, out_hbm.at[pl.ds(base, _TPW)])


@functools.lru_cache(maxsize=None)
def _sc_calls():
    mesh = plsc.VectorSubcoreMesh(core_axis_name="c", subcore_axis_name="s")
    dispatch = pl.kernel(
        _dispatch_body,
        out_type=[jax.ShapeDtypeStruct((_TP, _D), jnp.float32),
                  jax.ShapeDtypeStruct((_TP, 128), jnp.float32)],
        mesh=mesh,
        scratch_types=[pltpu.VMEM((_TPW,), jnp.int32),
                       pltpu.VMEM((_TPW, _D), jnp.float32),
                       pltpu.VMEM((_TPW, 128), jnp.float32),
                       pltpu.SemaphoreType.DMA,
                       pltpu.SemaphoreType.DMA])
    combine = pl.kernel(
        _combine_body,
        out_type=jax.ShapeDtypeStruct((_T, _D), jnp.float32),
        mesh=mesh,
        scratch_types=[pltpu.VMEM((_TPW,), jnp.int32),
                       pltpu.VMEM((_TPW, _D), jnp.float32),
                       pltpu.SemaphoreType.DMA])
    return dispatch, combine


_route_call = pl.pallas_call(
    _route_body,
    out_shape=[jax.ShapeDtypeStruct((_T, 1), jnp.int32),
               jax.ShapeDtypeStruct((_T, 128), jnp.float32),
               jax.ShapeDtypeStruct((1, _GPAD), jnp.int32)],
)

_ffn_call = pl.pallas_call(
    _ffn_body,
    grid_spec=pltpu.PrefetchScalarGridSpec(
        num_scalar_prefetch=1,
        grid=(_NF, _G),
        in_specs=[
            pl.BlockSpec((_BT, _D), lambda f, g, be: (g, 0)),
            pl.BlockSpec((_BT, 128), lambda f, g, be: (g, 0)),
            pl.BlockSpec((1, 1, _D), lambda f, g, be: (be[g], 0, 0)),
            pl.BlockSpec((1, 1, _D), lambda f, g, be: (be[g], 0, 0)),
            pl.BlockSpec((1, _D, _FC), lambda f, g, be: (be[g], 0, f)),
            pl.BlockSpec((1, 1, _FC), lambda f, g, be: (be[g], 0, f)),
            pl.BlockSpec((1, _FC, _D), lambda f, g, be: (be[g], f, 0)),
            pl.BlockSpec((1, 1, _D), lambda f, g, be: (be[g], 0, 0)),
        ],
        out_specs=pl.BlockSpec(
            (_BT, _D), lambda f, g, be: (jnp.where(f == _NF - 1, g, 0), 0)),
        scratch_shapes=[pltpu.VMEM((_TP, _D), jnp.float32),
                        pltpu.VMEM((_D, _FC), jnp.bfloat16),
                        pltpu.VMEM((_FC, _D), jnp.bfloat16)],
    ),
    out_shape=jax.ShapeDtypeStruct((_TP, _D), jnp.float32),
)


def kernel(input_features, input_ids, expert_centroids, ln_scale, ln_bias,
           W1, b1, W2, b2):
    s, b, d = input_features.shape
    x = input_features.reshape(s * b, d)
    dest2, alpha16, bexp2 = _route_call(x, expert_centroids)
    dest = dest2.reshape(_T)
    bexp = bexp2.reshape(_GPAD)
    dispatch, combine = _sc_calls()
    xs, als = dispatch(x, dest, alpha16)
    y = _ffn_call(bexp, xs, als,
                  ln_scale.reshape(_E, 1, _D), ln_bias.reshape(_E, 1, _D),
                  W1, b1.reshape(_E, 1, _F),
                  W2, b2.reshape(_E, 1, _D))
    out = combine(dest, y)
    return out.reshape(s, b, d)


# speedup vs baseline: 2.8777x; 1.1156x over previous
import functools

import jax
import jax.numpy as jnp
from jax import lax
from jax.experimental import pallas as pl
from jax.experimental.pallas import tpu as pltpu
from jax.experimental.pallas import tpu_sc as plsc

_E, _D, _F = 8, 1024, 4096
_T = 2048
_BT = 128
_G = _T // _BT + _E
_TP = _G * _BT
_NW = 32
_TPW = _T // _NW
_GPAD = 128
_NF = 2
_FC = _F // _NF


def _route_body(x_ref, c_ref, dest_ref, alpha_ref, bexp_ref, nblk_ref):
    x = x_ref[...]
    c = c_ref[...]
    aff = lax.dot_general(
        x, c, (((1,), (1,)), ((), ())),
        preferred_element_type=jnp.float32,
        precision=lax.Precision.DEFAULT)
    mx = jnp.max(aff, axis=1, keepdims=True)
    alpha = 1.0 / (1.0 + jnp.exp(-mx))
    alpha_ref[...] = jnp.broadcast_to(alpha, (_T, 128))
    eid = lax.broadcasted_iota(jnp.int32, (_T, _E), 1)
    idx = jnp.min(jnp.where(aff == mx, eid, _E), axis=1, keepdims=True)
    onehot = (eid == idx).astype(jnp.float32)
    tri = (lax.broadcasted_iota(jnp.int32, (_T, _T), 1)
           <= lax.broadcasted_iota(jnp.int32, (_T, _T), 0)).astype(jnp.float32)
    cum = lax.dot_general(tri, onehot, (((1,), (0,)), ((), ())),
                          preferred_element_type=jnp.float32)
    counts = cum[_T - 1:_T, :]
    rank = jnp.sum(cum * onehot, axis=1, keepdims=True) - 1.0
    nblk = jnp.ceil(counts * (1.0 / _BT))
    nb = jnp.broadcast_to(nblk, (_E, _E))
    strict_lo = (lax.broadcasted_iota(jnp.int32, (_E, _E), 1)
                 < lax.broadcasted_iota(jnp.int32, (_E, _E), 0)).astype(jnp.float32)
    blk_start = jnp.sum(nb * strict_lo, axis=1, keepdims=True)
    pad_start = blk_start * float(_BT)
    dest_base = lax.dot_general(onehot, pad_start, (((1,), (0,)), ((), ())),
                                preferred_element_type=jnp.float32)
    dest_ref[...] = (dest_base + rank).astype(jnp.int32)
    ge = (jnp.broadcast_to(blk_start, (_E, _GPAD))
          <= lax.broadcasted_iota(jnp.int32, (_E, _GPAD), 1).astype(jnp.float32)
          ).astype(jnp.float32)
    bexp_ref[...] = jnp.sum(ge, axis=0, keepdims=True).astype(jnp.int32) - 1
    nblk_ref[...] = nblk.astype(jnp.int32)


def _ffn_body(bexp_ref, nblk_ref, xs_ref, al_ref, lns_ref, lnb_ref,
              w1_any, b1_ref, w2_any, b2_ref, y_ref,
              acc_ref, w1raw_ref, w2raw_ref, w1b_ref, w2b_ref,
              slot_ref, sems):
    f = pl.program_id(0)
    g = pl.program_id(1)
    e = bexp_ref[0, g]
    eprev = bexp_ref[0, jnp.maximum(g - 1, 0)]

    def _fetch(slot, ee):
        pltpu.make_async_copy(
            w1_any.at[ee, :, pl.ds(f * _FC, _FC)],
            w1raw_ref.at[slot], sems.at[slot]).start()
        pltpu.make_async_copy(
            w2_any.at[ee, pl.ds(f * _FC, _FC), :],
            w2raw_ref.at[slot], sems.at[slot]).start()

    def _wait(slot):
        pltpu.make_async_copy(
            w1_any.at[0, :, pl.ds(0, _FC)],
            w1raw_ref.at[slot], sems.at[slot]).wait()
        pltpu.make_async_copy(
            w2_any.at[0, pl.ds(0, _FC), :],
            w2raw_ref.at[slot], sems.at[slot]).wait()

    @pl.when((g == 0) | (e != eprev))
    def _swap_weights():
        @pl.when(g == 0)
        def _init():
            slot_ref[0] = 0
            _fetch(0, e)

        s = slot_ref[0]
        _wait(s)
        w1b_ref[...] = w1raw_ref[s].astype(jnp.bfloat16)
        w2b_ref[...] = w2raw_ref[s].astype(jnp.bfloat16)
        ne = bexp_ref[0, g + nblk_ref[0, e]]

        @pl.when(ne != e)
        def _prefetch_next():
            _fetch(1 - s, ne)
            slot_ref[0] = 1 - s

    xs = xs_ref[...]
    mu = jnp.mean(xs, axis=1, keepdims=True)
    var = jnp.mean((xs - mu) * (xs - mu), axis=1, keepdims=True)
    hn = (xs - mu) * lax.rsqrt(var + 1e-5) * lns_ref[0] + lnb_ref[0]
    h = jnp.maximum(
        lax.dot_general(hn.astype(jnp.bfloat16), w1b_ref[...],
                        (((1,), (0,)), ((), ())),
                        preferred_element_type=jnp.float32) + b1_ref[0], 0.0)
    part = lax.dot_general(h.astype(jnp.bfloat16), w2b_ref[...],
                           (((1,), (0,)), ((), ())),
                           preferred_element_type=jnp.float32)

    @pl.when(f == 0)
    def _store_partial():
        acc_ref[pl.ds(g * _BT, _BT), :] = part

    @pl.when((f > 0) & (f < _NF - 1))
    def _add_partial():
        acc_ref[pl.ds(g * _BT, _BT), :] += part

    @pl.when(f == _NF - 1)
    def _finish():
        y_ref[...] = xs + al_ref[:, :1] * (
            acc_ref[pl.ds(g * _BT, _BT), :] + part + b2_ref[0])


def _dispatch_body(x_hbm, dest_hbm, alpha_hbm, xs_hbm, as_hbm,
                   idx_v, rows_v, al_v, sem1, sem2):
    wid = lax.axis_index("s") * 2 + lax.axis_index("c")
    base = wid * _TPW
    pltpu.sync_copy(dest_hbm.at[pl.ds(base, _TPW)], idx_v)
    pltpu.sync_copy(x_hbm.at[pl.ds(base, _TPW)], rows_v)
    pltpu.sync_copy(alpha_hbm.at[pl.ds(base, _TPW)], al_v)
    cp1 = pltpu.async_copy(rows_v, xs_hbm.at[idx_v], sem1)
    cp2 = pltpu.async_copy(al_v, as_hbm.at[idx_v], sem2)
    cp1.wait()
    cp2.wait()


def _combine_body(dest_hbm, y_hbm, out_hbm, idx_v, rows_v, sem):
    wid = lax.axis_index("s") * 2 + lax.axis_index("c")
    base = wid * _TPW
    pltpu.sync_copy(dest_hbm.at[pl.ds(base, _TPW)], idx_v)
    pltpu.async_copy(y_hbm.at[idx_v], rows_v, sem).wait()
    pltpu.sync_copy(rows_v, out_hbm.at[pl.ds(base, _TPW)])


@functools.lru_cache(maxsize=None)
def _sc_calls():
    mesh = plsc.VectorSubcoreMesh(core_axis_name="c", subcore_axis_name="s")
    dispatch = pl.kernel(
        _dispatch_body,
        out_type=[jax.ShapeDtypeStruct((_TP, _D), jnp.float32),
                  jax.ShapeDtypeStruct((_TP, 128), jnp.float32)],
        mesh=mesh,
        scratch_types=[pltpu.VMEM((_TPW,), jnp.int32),
                       pltpu.VMEM((_TPW, _D), jnp.float32),
                       pltpu.VMEM((_TPW, 128), jnp.float32),
                       pltpu.SemaphoreType.DMA,
                       pltpu.SemaphoreType.DMA])
    combine = pl.kernel(
        _combine_body,
        out_type=jax.ShapeDtypeStruct((_T, _D), jnp.float32),
        mesh=mesh,
        scratch_types=[pltpu.VMEM((_TPW,), jnp.int32),
                       pltpu.VMEM((_TPW, _D), jnp.float32),
                       pltpu.SemaphoreType.DMA])
    return dispatch, combine


_route_call = pl.pallas_call(
    _route_body,
    out_shape=[jax.ShapeDtypeStruct((_T, 1), jnp.int32),
               jax.ShapeDtypeStruct((_T, 128), jnp.float32),
               jax.ShapeDtypeStruct((1, _GPAD), jnp.int32),
               jax.ShapeDtypeStruct((1, _E), jnp.int32)],
)

_ffn_call = pl.pallas_call(
    _ffn_body,
    grid_spec=pltpu.PrefetchScalarGridSpec(
        num_scalar_prefetch=2,
        grid=(_NF, _G),
        in_specs=[
            pl.BlockSpec((_BT, _D), lambda f, g, be, nb: (g, 0)),
            pl.BlockSpec((_BT, 128), lambda f, g, be, nb: (g, 0)),
            pl.BlockSpec((1, 1, _D), lambda f, g, be, nb: (be[0, g], 0, 0)),
            pl.BlockSpec((1, 1, _D), lambda f, g, be, nb: (be[0, g], 0, 0)),
            pl.BlockSpec(memory_space=pl.ANY),
            pl.BlockSpec((1, 1, _FC), lambda f, g, be, nb: (be[0, g], 0, f)),
            pl.BlockSpec(memory_space=pl.ANY),
            pl.BlockSpec((1, 1, _D), lambda f, g, be, nb: (be[0, g], 0, 0)),
        ],
        out_specs=pl.BlockSpec(
            (_BT, _D), lambda f, g, be, nb: (jnp.where(f == _NF - 1, g, 0), 0)),
        scratch_shapes=[pltpu.VMEM((_TP, _D), jnp.float32),
                        pltpu.VMEM((2, _D, _FC), jnp.float32),
                        pltpu.VMEM((2, _FC, _D), jnp.float32),
                        pltpu.VMEM((_D, _FC), jnp.bfloat16),
                        pltpu.VMEM((_FC, _D), jnp.bfloat16),
                        pltpu.SMEM((1,), jnp.int32),
                        pltpu.SemaphoreType.DMA((2,))],
    ),
    out_shape=jax.ShapeDtypeStruct((_TP, _D), jnp.float32),
    compiler_params=pltpu.CompilerParams(vmem_limit_bytes=62 * 1024 * 1024),
)


def kernel(input_features, input_ids, expert_centroids, ln_scale, ln_bias,
           W1, b1, W2, b2):
    s, b, d = input_features.shape
    x = input_features.reshape(s * b, d)
    dest2, alpha16, bexp2, nblk2 = _route_call(x, expert_centroids)
    dest = dest2.reshape(_T)
    dispatch, combine = _sc_calls()
    xs, als = dispatch(x, dest, alpha16)
    y = _ffn_call(bexp2, nblk2, xs, als,
                  ln_scale.reshape(_E, 1, _D), ln_bias.reshape(_E, 1, _D),
                  W1, b1.reshape(_E, 1, _F),
                  W2, b2.reshape(_E, 1, _D))
    out = combine(dest, y)
    return out.reshape(s, b, d)


# speedup vs baseline: 2.9697x; 1.0320x over previous
import functools

import jax
import jax.numpy as jnp
from jax import lax
from jax.experimental import pallas as pl
from jax.experimental.pallas import tpu as pltpu
from jax.experimental.pallas import tpu_sc as plsc

_E, _D, _F = 8, 1024, 4096
_T = 2048
_BT = 128
_G = _T // _BT + _E
_TP = _G * _BT
_NW = 32
_TPW = _T // _NW
_GPAD = 128
_NF = 2
_FC = _F // _NF


def _route_body(x_ref, c_ref, dest_ref, alpha_ref, bexp_ref, nblk_ref):
    x = x_ref[...]
    c = c_ref[...]
    aff = lax.dot_general(
        x, c, (((1,), (1,)), ((), ())),
        preferred_element_type=jnp.float32,
        precision=lax.Precision.DEFAULT)
    mx = jnp.max(aff, axis=1, keepdims=True)
    alpha = 1.0 / (1.0 + jnp.exp(-mx))
    alpha_ref[...] = jnp.broadcast_to(alpha, (_T, 128))
    eid = lax.broadcasted_iota(jnp.int32, (_T, _E), 1)
    idx = jnp.min(jnp.where(aff == mx, eid, _E), axis=1, keepdims=True)
    onehot = (eid == idx).astype(jnp.float32)
    tri = (lax.broadcasted_iota(jnp.int32, (_T, _T), 1)
           <= lax.broadcasted_iota(jnp.int32, (_T, _T), 0)).astype(jnp.float32)
    cum = lax.dot_general(tri, onehot, (((1,), (0,)), ((), ())),
                          preferred_element_type=jnp.float32)
    counts = cum[_T - 1:_T, :]
    rank = jnp.sum(cum * onehot, axis=1, keepdims=True) - 1.0
    nblk = jnp.ceil(counts * (1.0 / _BT))
    nb = jnp.broadcast_to(nblk, (_E, _E))
    strict_lo = (lax.broadcasted_iota(jnp.int32, (_E, _E), 1)
                 < lax.broadcasted_iota(jnp.int32, (_E, _E), 0)).astype(jnp.float32)
    blk_start = jnp.sum(nb * strict_lo, axis=1, keepdims=True)
    pad_start = blk_start * float(_BT)
    dest_base = lax.dot_general(onehot, pad_start, (((1,), (0,)), ((), ())),
                                preferred_element_type=jnp.float32)
    dest_ref[...] = (dest_base + rank).astype(jnp.int32)
    ge = (jnp.broadcast_to(blk_start, (_E, _GPAD))
          <= lax.broadcasted_iota(jnp.int32, (_E, _GPAD), 1).astype(jnp.float32)
          ).astype(jnp.float32)
    bexp_ref[...] = jnp.sum(ge, axis=0, keepdims=True).astype(jnp.int32) - 1
    nblk_ref[...] = nblk.astype(jnp.int32)


def _ffn_body(bexp_ref, nblk_ref, xs_ref, al_ref, lns_ref, lnb_ref,
              w1_any, b1_ref, w2_any, b2_ref, y_ref,
              acc_ref, w1raw_ref, w2raw_ref, w1b_ref, w2b_ref,
              slot_ref, sems):
    f = pl.program_id(0)
    g = pl.program_id(1)
    e = bexp_ref[0, g]
    eprev = bexp_ref[0, jnp.maximum(g - 1, 0)]

    def _fetch(slot, ee):
        pltpu.make_async_copy(
            w1_any.at[ee, :, pl.ds(f * _FC, _FC)],
            w1raw_ref.at[slot], sems.at[slot]).start()
        pltpu.make_async_copy(
            w2_any.at[ee, pl.ds(f * _FC, _FC), :],
            w2raw_ref.at[slot], sems.at[slot]).start()

    def _wait(slot):
        pltpu.make_async_copy(
            w1_any.at[0, :, pl.ds(0, _FC)],
            w1raw_ref.at[slot], sems.at[slot]).wait()
        pltpu.make_async_copy(
            w2_any.at[0, pl.ds(0, _FC), :],
            w2raw_ref.at[slot], sems.at[slot]).wait()

    @pl.when((g == 0) | (e != eprev))
    def _swap_weights():
        @pl.when(g == 0)
        def _init():
            slot_ref[0] = 0
            _fetch(0, e)

        s = slot_ref[0]
        _wait(s)
        ne = bexp_ref[0, g + nblk_ref[0, e]]

        @pl.when(ne != e)
        def _prefetch_next():
            _fetch(1 - s, ne)
            slot_ref[0] = 1 - s

        w1b_ref[...] = w1raw_ref[s].astype(jnp.bfloat16)
        w2b_ref[...] = w2raw_ref[s].astype(jnp.bfloat16)

    xs = xs_ref[...]
    mu = jnp.mean(xs, axis=1, keepdims=True)
    var = jnp.mean((xs - mu) * (xs - mu), axis=1, keepdims=True)
    hn = (xs - mu) * lax.rsqrt(var + 1e-5) * lns_ref[0] + lnb_ref[0]
    h = jnp.maximum(
        lax.dot_general(hn.astype(jnp.bfloat16), w1b_ref[...],
                        (((1,), (0,)), ((), ())),
                        preferred_element_type=jnp.float32) + b1_ref[0], 0.0)
    part = lax.dot_general(h.astype(jnp.bfloat16), w2b_ref[...],
                           (((1,), (0,)), ((), ())),
                           preferred_element_type=jnp.float32)

    @pl.when(f == 0)
    def _store_partial():
        acc_ref[pl.ds(g * _BT, _BT), :] = part

    @pl.when((f > 0) & (f < _NF - 1))
    def _add_partial():
        acc_ref[pl.ds(g * _BT, _BT), :] += part

    @pl.when(f == _NF - 1)
    def _finish():
        y_ref[...] = xs + al_ref[:, :1] * (
            acc_ref[pl.ds(g * _BT, _BT), :] + part + b2_ref[0])


def _dispatch_body(x_hbm, dest_hbm, alpha_hbm, xs_hbm, as_hbm,
                   idx_v, rows_v, al_v, sem1, sem2):
    wid = lax.axis_index("s") * 2 + lax.axis_index("c")
    base = wid * _TPW
    pltpu.sync_copy(dest_hbm.at[pl.ds(base, _TPW)], idx_v)
    pltpu.sync_copy(x_hbm.at[pl.ds(base, _TPW)], rows_v)
    pltpu.sync_copy(alpha_hbm.at[pl.ds(base, _TPW)], al_v)
    cp1 = pltpu.async_copy(rows_v, xs_hbm.at[idx_v], sem1)
    cp2 = pltpu.async_copy(al_v, as_hbm.at[idx_v], sem2)
    cp1.wait()
    cp2.wait()


def _combine_body(dest_hbm, y_hbm, out_hbm, idx_v, rows_v, sem):
    wid = lax.axis_index("s") * 2 + lax.axis_index("c")
    base = wid * _TPW
    pltpu.sync_copy(dest_hbm.at[pl.ds(base, _TPW)], idx_v)
    pltpu.async_copy(y_hbm.at[idx_v], rows_v, sem).wait()
    pltpu.sync_copy(rows_v, out_hbm.at[pl.ds(base, _TPW)])


@functools.lru_cache(maxsize=None)
def _sc_calls():
    mesh = plsc.VectorSubcoreMesh(core_axis_name="c", subcore_axis_name="s")
    dispatch = pl.kernel(
        _dispatch_body,
        out_type=[jax.ShapeDtypeStruct((_TP, _D), jnp.float32),
                  jax.ShapeDtypeStruct((_TP, 128), jnp.float32)],
        mesh=mesh,
        scratch_types=[pltpu.VMEM((_TPW,), jnp.int32),
                       pltpu.VMEM((_TPW, _D), jnp.float32),
                       pltpu.VMEM((_TPW, 128), jnp.float32),
                       pltpu.SemaphoreType.DMA,
                       pltpu.SemaphoreType.DMA])
    combine = pl.kernel(
        _combine_body,
        out_type=jax.ShapeDtypeStruct((_T, _D), jnp.float32),
        mesh=mesh,
        scratch_types=[pltpu.VMEM((_TPW,), jnp.int32),
                       pltpu.VMEM((_TPW, _D), jnp.float32),
                       pltpu.SemaphoreType.DMA])
    return dispatch, combine


_route_call = pl.pallas_call(
    _route_body,
    out_shape=[jax.ShapeDtypeStruct((_T, 1), jnp.int32),
               jax.ShapeDtypeStruct((_T, 128), jnp.float32),
               jax.ShapeDtypeStruct((1, _GPAD), jnp.int32),
               jax.ShapeDtypeStruct((1, _E), jnp.int32)],
)

_ffn_call = pl.pallas_call(
    _ffn_body,
    grid_spec=pltpu.PrefetchScalarGridSpec(
        num_scalar_prefetch=2,
        grid=(_NF, _G),
        in_specs=[
            pl.BlockSpec((_BT, _D), lambda f, g, be, nb: (g, 0)),
            pl.BlockSpec((_BT, 128), lambda f, g, be, nb: (g, 0)),
            pl.BlockSpec((1, 1, _D), lambda f, g, be, nb: (be[0, g], 0, 0)),
            pl.BlockSpec((1, 1, _D), lambda f, g, be, nb: (be[0, g], 0, 0)),
            pl.BlockSpec(memory_space=pl.ANY),
            pl.BlockSpec((1, 1, _FC), lambda f, g, be, nb: (be[0, g], 0, f)),
            pl.BlockSpec(memory_space=pl.ANY),
            pl.BlockSpec((1, 1, _D), lambda f, g, be, nb: (be[0, g], 0, 0)),
        ],
        out_specs=pl.BlockSpec(
            (_BT, _D), lambda f, g, be, nb: (jnp.where(f == _NF - 1, g, 0), 0)),
        scratch_shapes=[pltpu.VMEM((_TP, _D), jnp.float32),
                        pltpu.VMEM((2, _D, _FC), jnp.float32),
                        pltpu.VMEM((2, _FC, _D), jnp.float32),
                        pltpu.VMEM((_D, _FC), jnp.bfloat16),
                        pltpu.VMEM((_FC, _D), jnp.bfloat16),
                        pltpu.SMEM((1,), jnp.int32),
                        pltpu.SemaphoreType.DMA((2,))],
    ),
    out_shape=jax.ShapeDtypeStruct((_TP, _D), jnp.float32),
    compiler_params=pltpu.CompilerParams(vmem_limit_bytes=62 * 1024 * 1024),
)


def kernel(input_features, input_ids, expert_centroids, ln_scale, ln_bias,
           W1, b1, W2, b2):
    s, b, d = input_features.shape
    x = input_features.reshape(s * b, d)
    dest2, alpha16, bexp2, nblk2 = _route_call(x, expert_centroids)
    dest = dest2.reshape(_T)
    dispatch, combine = _sc_calls()
    xs, als = dispatch(x, dest, alpha16)
    y = _ffn_call(bexp2, nblk2, xs, als,
                  ln_scale.reshape(_E, 1, _D), ln_bias.reshape(_E, 1, _D),
                  W1, b1.reshape(_E, 1, _F),
                  W2, b2.reshape(_E, 1, _D))
    out = combine(dest, y)
    return out.reshape(s, b, d)


# speedup vs baseline: 3.0006x; 1.0104x over previous
import functools

import jax
import jax.numpy as jnp
from jax import lax
from jax.experimental import pallas as pl
from jax.experimental.pallas import tpu as pltpu
from jax.experimental.pallas import tpu_sc as plsc

_E, _D, _F = 8, 1024, 4096
_T = 2048
_BT = 128
_G = _T // _BT + _E
_TP = _G * _BT
_NW = 32
_TPW = _T // _NW
_GPAD = 128
_NF = 2
_FC = _F // _NF


def _route_body(x_ref, c_ref, dest_ref, alpha_ref, bexp_ref, nblk_ref):
    x = x_ref[...]
    c = c_ref[...]
    aff = lax.dot_general(
        x, c, (((1,), (1,)), ((), ())),
        preferred_element_type=jnp.float32,
        precision=lax.Precision.DEFAULT)
    mx = jnp.max(aff, axis=1, keepdims=True)
    alpha = 1.0 / (1.0 + jnp.exp(-mx))
    alpha_ref[...] = jnp.broadcast_to(alpha, (_T, 128))
    eid = lax.broadcasted_iota(jnp.int32, (_T, _E), 1)
    idx = jnp.min(jnp.where(aff == mx, eid, _E), axis=1, keepdims=True)
    onehot = (eid == idx).astype(jnp.float32)
    tri = (lax.broadcasted_iota(jnp.int32, (_T, _T), 1)
           <= lax.broadcasted_iota(jnp.int32, (_T, _T), 0)).astype(jnp.float32)
    cum = lax.dot_general(tri, onehot, (((1,), (0,)), ((), ())),
                          preferred_element_type=jnp.float32)
    counts = cum[_T - 1:_T, :]
    rank = jnp.sum(cum * onehot, axis=1, keepdims=True) - 1.0
    nblk = jnp.ceil(counts * (1.0 / _BT))
    nb = jnp.broadcast_to(nblk, (_E, _E))
    strict_lo = (lax.broadcasted_iota(jnp.int32, (_E, _E), 1)
                 < lax.broadcasted_iota(jnp.int32, (_E, _E), 0)).astype(jnp.float32)
    blk_start = jnp.sum(nb * strict_lo, axis=1, keepdims=True)
    pad_start = blk_start * float(_BT)
    dest_base = lax.dot_general(onehot, pad_start, (((1,), (0,)), ((), ())),
                                preferred_element_type=jnp.float32)
    dest_ref[...] = (dest_base + rank).astype(jnp.int32).reshape(_T // 128, 128)
    ge = (jnp.broadcast_to(blk_start, (_E, _GPAD))
          <= lax.broadcasted_iota(jnp.int32, (_E, _GPAD), 1).astype(jnp.float32)
          ).astype(jnp.float32)
    bexp_ref[...] = jnp.sum(ge, axis=0, keepdims=True).astype(jnp.int32) - 1
    nblk_ref[...] = nblk.astype(jnp.int32)


def _ffn_body(bexp_ref, nblk_ref, xs_ref, al_ref, lns_ref, lnb_ref,
              w1_any, b1_ref, w2_any, b2_ref, y_ref,
              acc_ref, w1raw_ref, w2raw_ref, w1b_ref, w2b_ref,
              slot_ref, sems):
    f = pl.program_id(0)
    g = pl.program_id(1)
    e = bexp_ref[0, g]
    eprev = bexp_ref[0, jnp.maximum(g - 1, 0)]

    def _fetch(slot, ee):
        pltpu.make_async_copy(
            w1_any.at[ee, :, pl.ds(f * _FC, _FC)],
            w1raw_ref.at[slot], sems.at[slot]).start()
        pltpu.make_async_copy(
            w2_any.at[ee, pl.ds(f * _FC, _FC), :],
            w2raw_ref.at[slot], sems.at[slot]).start()

    def _wait(slot):
        pltpu.make_async_copy(
            w1_any.at[0, :, pl.ds(0, _FC)],
            w1raw_ref.at[slot], sems.at[slot]).wait()
        pltpu.make_async_copy(
            w2_any.at[0, pl.ds(0, _FC), :],
            w2raw_ref.at[slot], sems.at[slot]).wait()

    @pl.when((g == 0) | (e != eprev))
    def _swap_weights():
        @pl.when(g == 0)
        def _init():
            slot_ref[0] = 0
            _fetch(0, e)

        s = slot_ref[0]
        _wait(s)
        ne = bexp_ref[0, g + nblk_ref[0, e]]

        @pl.when(ne != e)
        def _prefetch_next():
            _fetch(1 - s, ne)
            slot_ref[0] = 1 - s

        w1b_ref[...] = w1raw_ref[s].astype(jnp.bfloat16)
        w2b_ref[...] = w2raw_ref[s].astype(jnp.bfloat16)

    xs = xs_ref[...]
    mu = jnp.mean(xs, axis=1, keepdims=True)
    var = jnp.mean((xs - mu) * (xs - mu), axis=1, keepdims=True)
    hn = (xs - mu) * lax.rsqrt(var + 1e-5) * lns_ref[0] + lnb_ref[0]
    h = jnp.maximum(
        lax.dot_general(hn.astype(jnp.bfloat16), w1b_ref[...],
                        (((1,), (0,)), ((), ())),
                        preferred_element_type=jnp.float32) + b1_ref[0], 0.0)
    part = lax.dot_general(h.astype(jnp.bfloat16), w2b_ref[...],
                           (((1,), (0,)), ((), ())),
                           preferred_element_type=jnp.float32)

    @pl.when(f == 0)
    def _store_partial():
        acc_ref[pl.ds(g * _BT, _BT), :] = part

    @pl.when((f > 0) & (f < _NF - 1))
    def _add_partial():
        acc_ref[pl.ds(g * _BT, _BT), :] += part

    @pl.when(f == _NF - 1)
    def _finish():
        y_ref[...] = xs + al_ref[:, :1] * (
            acc_ref[pl.ds(g * _BT, _BT), :] + part + b2_ref[0])


def _dispatch_body(x_hbm, dest_hbm, alpha_hbm, xs_hbm, as_hbm,
                   idx_v, rows_v, al_v, sem1, sem2):
    wid = lax.axis_index("s") * 2 + lax.axis_index("c")
    base = wid * _TPW
    pltpu.sync_copy(dest_hbm.at[pl.ds(base, _TPW)], idx_v)
    pltpu.sync_copy(x_hbm.at[pl.ds(base, _TPW)], rows_v)
    pltpu.sync_copy(alpha_hbm.at[pl.ds(base, _TPW)], al_v)
    cp1 = pltpu.async_copy(rows_v, xs_hbm.at[idx_v], sem1)
    cp2 = pltpu.async_copy(al_v, as_hbm.at[idx_v], sem2)
    cp1.wait()
    cp2.wait()


def _combine_body(dest_hbm, y_hbm, out_hbm, idx_v, rows_v, sem):
    wid = lax.axis_index("s") * 2 + lax.axis_index("c")
    base = wid * _TPW
    pltpu.sync_copy(dest_hbm.at[pl.ds(base, _TPW)], idx_v)
    pltpu.async_copy(y_hbm.at[idx_v], rows_v, sem).wait()
    pltpu.sync_copy(rows_v, out_hbm.at[pl.ds(base, _TPW)])


@functools.lru_cache(maxsize=None)
def _sc_calls():
    mesh = plsc.VectorSubcoreMesh(core_axis_name="c", subcore_axis_name="s")
    dispatch = pl.kernel(
        _dispatch_body,
        out_type=[jax.ShapeDtypeStruct((_TP, _D), jnp.float32),
                  jax.ShapeDtypeStruct((_TP, 128), jnp.float32)],
        mesh=mesh,
        scratch_types=[pltpu.VMEM((_TPW,), jnp.int32),
                       pltpu.VMEM((_TPW, _D), jnp.float32),
                       pltpu.VMEM((_TPW, 128), jnp.float32),
                       pltpu.SemaphoreType.DMA,
                       pltpu.SemaphoreType.DMA])
    combine = pl.kernel(
        _combine_body,
        out_type=jax.ShapeDtypeStruct((_T, _D), jnp.float32),
        mesh=mesh,
        scratch_types=[pltpu.VMEM((_TPW,), jnp.int32),
                       pltpu.VMEM((_TPW, _D), jnp.float32),
                       pltpu.SemaphoreType.DMA])
    return dispatch, combine


_route_call = pl.pallas_call(
    _route_body,
    out_shape=[jax.ShapeDtypeStruct((_T // 128, 128), jnp.int32),
               jax.ShapeDtypeStruct((_T, 128), jnp.float32),
               jax.ShapeDtypeStruct((1, _GPAD), jnp.int32),
               jax.ShapeDtypeStruct((1, _E), jnp.int32)],
)

_ffn_call = pl.pallas_call(
    _ffn_body,
    grid_spec=pltpu.PrefetchScalarGridSpec(
        num_scalar_prefetch=2,
        grid=(_NF, _G),
        in_specs=[
            pl.BlockSpec((_BT, _D), lambda f, g, be, nb: (g, 0)),
            pl.BlockSpec((_BT, 128), lambda f, g, be, nb: (g, 0)),
            pl.BlockSpec((1, 1, _D), lambda f, g, be, nb: (be[0, g], 0, 0)),
            pl.BlockSpec((1, 1, _D), lambda f, g, be, nb: (be[0, g], 0, 0)),
            pl.BlockSpec(memory_space=pl.ANY),
            pl.BlockSpec((1, 1, _FC), lambda f, g, be, nb: (be[0, g], 0, f)),
            pl.BlockSpec(memory_space=pl.ANY),
            pl.BlockSpec((1, 1, _D), lambda f, g, be, nb: (be[0, g], 0, 0)),
        ],
        out_specs=pl.BlockSpec(
            (_BT, _D), lambda f, g, be, nb: (jnp.where(f == _NF - 1, g, 0), 0)),
        scratch_shapes=[pltpu.VMEM((_TP, _D), jnp.float32),
                        pltpu.VMEM((2, _D, _FC), jnp.float32),
                        pltpu.VMEM((2, _FC, _D), jnp.float32),
                        pltpu.VMEM((_D, _FC), jnp.bfloat16),
                        pltpu.VMEM((_FC, _D), jnp.bfloat16),
                        pltpu.SMEM((1,), jnp.int32),
                        pltpu.SemaphoreType.DMA((2,))],
    ),
    out_shape=jax.ShapeDtypeStruct((_TP, _D), jnp.float32),
    compiler_params=pltpu.CompilerParams(vmem_limit_bytes=62 * 1024 * 1024),
)


def kernel(input_features, input_ids, expert_centroids, ln_scale, ln_bias,
           W1, b1, W2, b2):
    s, b, d = input_features.shape
    x = input_features.reshape(s * b, d)
    dest2, alpha16, bexp2, nblk2 = _route_call(x, expert_centroids)
    dest = dest2.reshape(_T)
    dispatch, combine = _sc_calls()
    xs, als = dispatch(x, dest, alpha16)
    y = _ffn_call(bexp2, nblk2, xs, als,
                  ln_scale.reshape(_E, 1, _D), ln_bias.reshape(_E, 1, _D),
                  W1, b1.reshape(_E, 1, _F),
                  W2, b2.reshape(_E, 1, _D))
    out = combine(dest, y)
    return out.reshape(s, b, d)


# speedup vs baseline: 3.0226x; 1.0073x over previous
import functools

import jax
import jax.numpy as jnp
from jax import lax
from jax.experimental import pallas as pl
from jax.experimental.pallas import tpu as pltpu
from jax.experimental.pallas import tpu_sc as plsc

_E, _D, _F = 8, 1024, 4096
_T = 2048
_BT = 128
_G = _T // _BT + _E
_TP = _G * _BT
_NW = 32
_TPW = _T // _NW
_GPAD = 128
_NF = 2
_FC = _F // _NF


def _route_body(x_ref, c_ref, dest_ref, alpha_ref, bexp_ref, nblk_ref):
    x = x_ref[...]
    c = c_ref[...]
    aff = lax.dot_general(
        x, c, (((1,), (1,)), ((), ())),
        preferred_element_type=jnp.float32,
        precision=lax.Precision.DEFAULT)
    mx = jnp.max(aff, axis=1, keepdims=True)
    alpha = 1.0 / (1.0 + jnp.exp(-mx))
    alpha_ref[...] = jnp.broadcast_to(alpha, (_T, 128))
    eid = lax.broadcasted_iota(jnp.int32, (_T, _E), 1)
    idx = jnp.min(jnp.where(aff == mx, eid, _E), axis=1, keepdims=True)
    onehot = (eid == idx).astype(jnp.float32)
    tri = (lax.broadcasted_iota(jnp.int32, (_T, _T), 1)
           <= lax.broadcasted_iota(jnp.int32, (_T, _T), 0)).astype(jnp.float32)
    cum = lax.dot_general(tri, onehot, (((1,), (0,)), ((), ())),
                          preferred_element_type=jnp.float32)
    counts = cum[_T - 1:_T, :]
    rank = jnp.sum(cum * onehot, axis=1, keepdims=True) - 1.0
    nblk = jnp.ceil(counts * (1.0 / _BT))
    nb = jnp.broadcast_to(nblk, (_E, _E))
    strict_lo = (lax.broadcasted_iota(jnp.int32, (_E, _E), 1)
                 < lax.broadcasted_iota(jnp.int32, (_E, _E), 0)).astype(jnp.float32)
    blk_start = jnp.sum(nb * strict_lo, axis=1, keepdims=True)
    pad_start = blk_start * float(_BT)
    dest_base = lax.dot_general(onehot, pad_start, (((1,), (0,)), ((), ())),
                                preferred_element_type=jnp.float32)
    dest_ref[...] = (dest_base + rank).astype(jnp.int32).reshape(_T // 128, 128)
    ge = (jnp.broadcast_to(blk_start, (_E, _GPAD))
          <= lax.broadcasted_iota(jnp.int32, (_E, _GPAD), 1).astype(jnp.float32)
          ).astype(jnp.float32)
    bexp_ref[...] = jnp.sum(ge, axis=0, keepdims=True).astype(jnp.int32) - 1
    nblk_ref[...] = nblk.astype(jnp.int32)


def _ffn_body(bexp_ref, nblk_ref, xs_ref, al_ref, lns_ref, lnb_ref,
              w1_any, b1_ref, w2_any, b2_ref, y_ref,
              acc_ref, w1raw_ref, w2raw_ref,
              slot_ref, sems):
    f = pl.program_id(0)
    g = pl.program_id(1)
    e = bexp_ref[0, g]
    eprev = bexp_ref[0, jnp.maximum(g - 1, 0)]

    def _fetch(slot, ee):
        pltpu.make_async_copy(
            w1_any.at[ee, :, pl.ds(f * _FC, _FC)],
            w1raw_ref.at[slot], sems.at[slot]).start()
        pltpu.make_async_copy(
            w2_any.at[ee, pl.ds(f * _FC, _FC), :],
            w2raw_ref.at[slot], sems.at[slot]).start()

    def _wait(slot):
        pltpu.make_async_copy(
            w1_any.at[0, :, pl.ds(0, _FC)],
            w1raw_ref.at[slot], sems.at[slot]).wait()
        pltpu.make_async_copy(
            w2_any.at[0, pl.ds(0, _FC), :],
            w2raw_ref.at[slot], sems.at[slot]).wait()

    @pl.when(g == 0)
    def _init():
        slot_ref[0] = 0
        _fetch(0, e)

    @pl.when((g > 0) & (e != eprev))
    def _flip():
        slot_ref[0] = 1 - slot_ref[0]

    s = slot_ref[0]

    @pl.when((g == 0) | (e != eprev))
    def _swap_weights():
        _wait(s)
        ne = bexp_ref[0, g + nblk_ref[0, e]]

        @pl.when(ne != e)
        def _prefetch_next():
            _fetch(1 - s, ne)

    xs = xs_ref[...]
    mu = jnp.mean(xs, axis=1, keepdims=True)
    var = jnp.mean((xs - mu) * (xs - mu), axis=1, keepdims=True)
    hn = (xs - mu) * lax.rsqrt(var + 1e-5) * lns_ref[0] + lnb_ref[0]
    h = jnp.maximum(
        lax.dot_general(hn, w1raw_ref[s], (((1,), (0,)), ((), ())),
                        preferred_element_type=jnp.float32) + b1_ref[0], 0.0)
    part = lax.dot_general(h, w2raw_ref[s], (((1,), (0,)), ((), ())),
                           preferred_element_type=jnp.float32)

    @pl.when(f == 0)
    def _store_partial():
        acc_ref[pl.ds(g * _BT, _BT), :] = part

    @pl.when((f > 0) & (f < _NF - 1))
    def _add_partial():
        acc_ref[pl.ds(g * _BT, _BT), :] += part

    @pl.when(f == _NF - 1)
    def _finish():
        y_ref[...] = xs + al_ref[:, :1] * (
            acc_ref[pl.ds(g * _BT, _BT), :] + part + b2_ref[0])


def _dispatch_body(x_hbm, dest_hbm, alpha_hbm, xs_hbm, as_hbm,
                   idx_v, rows_v, al_v, sem1, sem2):
    wid = lax.axis_index("s") * 2 + lax.axis_index("c")
    base = wid * _TPW
    pltpu.sync_copy(dest_hbm.at[pl.ds(base, _TPW)], idx_v)
    pltpu.sync_copy(x_hbm.at[pl.ds(base, _TPW)], rows_v)
    pltpu.sync_copy(alpha_hbm.at[pl.ds(base, _TPW)], al_v)
    cp1 = pltpu.async_copy(rows_v, xs_hbm.at[idx_v], sem1)
    cp2 = pltpu.async_copy(al_v, as_hbm.at[idx_v], sem2)
    cp1.wait()
    cp2.wait()


def _combine_body(dest_hbm, y_hbm, out_hbm, idx_v, rows_v, sem):
    wid = lax.axis_index("s") * 2 + lax.axis_index("c")
    base = wid * _TPW
    pltpu.sync_copy(dest_hbm.at[pl.ds(base, _TPW)], idx_v)
    pltpu.async_copy(y_hbm.at[idx_v], rows_v, sem).wait()
    pltpu.sync_copy(rows_v, out_hbm.at[pl.ds(base, _TPW)])


@functools.lru_cache(maxsize=None)
def _sc_calls():
    mesh = plsc.VectorSubcoreMesh(core_axis_name="c", subcore_axis_name="s")
    dispatch = pl.kernel(
        _dispatch_body,
        out_type=[jax.ShapeDtypeStruct((_TP, _D), jnp.float32),
                  jax.ShapeDtypeStruct((_TP, 128), jnp.float32)],
        mesh=mesh,
        scratch_types=[pltpu.VMEM((_TPW,), jnp.int32),
                       pltpu.VMEM((_TPW, _D), jnp.float32),
                       pltpu.VMEM((_TPW, 128), jnp.float32),
                       pltpu.SemaphoreType.DMA,
                       pltpu.SemaphoreType.DMA])
    combine = pl.kernel(
        _combine_body,
        out_type=jax.ShapeDtypeStruct((_T, _D), jnp.float32),
        mesh=mesh,
        scratch_types=[pltpu.VMEM((_TPW,), jnp.int32),
                       pltpu.VMEM((_TPW, _D), jnp.float32),
                       pltpu.SemaphoreType.DMA])
    return dispatch, combine


_route_call = pl.pallas_call(
    _route_body,
    out_shape=[jax.ShapeDtypeStruct((_T // 128, 128), jnp.int32),
               jax.ShapeDtypeStruct((_T, 128), jnp.float32),
               jax.ShapeDtypeStruct((1, _GPAD), jnp.int32),
               jax.ShapeDtypeStruct((1, _E), jnp.int32)],
)

_ffn_call = pl.pallas_call(
    _ffn_body,
    grid_spec=pltpu.PrefetchScalarGridSpec(
        num_scalar_prefetch=2,
        grid=(_NF, _G),
        in_specs=[
            pl.BlockSpec((_BT, _D), lambda f, g, be, nb: (g, 0)),
            pl.BlockSpec((_BT, 128), lambda f, g, be, nb: (g, 0)),
            pl.BlockSpec((1, 1, _D), lambda f, g, be, nb: (be[0, g], 0, 0)),
            pl.BlockSpec((1, 1, _D), lambda f, g, be, nb: (be[0, g], 0, 0)),
            pl.BlockSpec(memory_space=pl.ANY),
            pl.BlockSpec((1, 1, _FC), lambda f, g, be, nb: (be[0, g], 0, f)),
            pl.BlockSpec(memory_space=pl.ANY),
            pl.BlockSpec((1, 1, _D), lambda f, g, be, nb: (be[0, g], 0, 0)),
        ],
        out_specs=pl.BlockSpec(
            (_BT, _D), lambda f, g, be, nb: (jnp.where(f == _NF - 1, g, 0), 0)),
        scratch_shapes=[pltpu.VMEM((_TP, _D), jnp.float32),
                        pltpu.VMEM((2, _D, _FC), jnp.float32),
                        pltpu.VMEM((2, _FC, _D), jnp.float32),
                        pltpu.SMEM((1,), jnp.int32),
                        pltpu.SemaphoreType.DMA((2,))],
    ),
    out_shape=jax.ShapeDtypeStruct((_TP, _D), jnp.float32),
    compiler_params=pltpu.CompilerParams(vmem_limit_bytes=62 * 1024 * 1024),
)


def kernel(input_features, input_ids, expert_centroids, ln_scale, ln_bias,
           W1, b1, W2, b2):
    s, b, d = input_features.shape
    x = input_features.reshape(s * b, d)
    dest2, alpha16, bexp2, nblk2 = _route_call(x, expert_centroids)
    dest = dest2.reshape(_T)
    dispatch, combine = _sc_calls()
    xs, als = dispatch(x, dest, alpha16)
    y = _ffn_call(bexp2, nblk2, xs, als,
                  ln_scale.reshape(_E, 1, _D), ln_bias.reshape(_E, 1, _D),
                  W1, b1.reshape(_E, 1, _F),
                  W2, b2.reshape(_E, 1, _D))
    out = combine(dest, y)
    return out.reshape(s, b, d)
